# baseline (speedup 1.0000x reference)
import functools

import jax
import jax.numpy as jnp
from jax.experimental import pallas as pl

_PACK = 16
_TILE = 2048


def _mlp_kernel(x_ref, w1_ref, b1_ref, w2_ref, b2_ref, w3_ref, b3_ref,
                w4_ref, b4_ref, o_ref):
    x = x_ref[...]
    h = jnp.dot(x, w1_ref[...], preferred_element_type=jnp.float32) + b1_ref[...]
    h = jnp.dot(h, w2_ref[...], preferred_element_type=jnp.float32) + b2_ref[...]
    h = jnp.maximum(h, 0.0)
    h = jnp.dot(h, w3_ref[...], preferred_element_type=jnp.float32) + b3_ref[...]
    h = jnp.maximum(h, 0.0)
    h = jnp.dot(h, w4_ref[...], preferred_element_type=jnp.float32) + b4_ref[...]
    o_ref[...] = jnp.maximum(h, 0.0)


def _softmax_topk_kernel(s_ref, idx_ref, *, n, k, topk):
    x = s_ref[0]
    m = jnp.max(x, axis=1, keepdims=True)
    p = jnp.exp(x - m)
    denom = jnp.sum(p, axis=1, keepdims=True)
    probs = p / denom
    kiota = jax.lax.broadcasted_iota(jnp.int32, (k, n), 0)
    rows = []
    a = probs
    for _ in range(topk):
        mx = jnp.max(a, axis=0, keepdims=True)
        sel = jnp.min(jnp.where(a == mx, kiota, k), axis=0, keepdims=True)
        rows.append(sel)
        a = jnp.where(kiota == sel, -1.0, a)
    idx_ref[0] = jnp.concatenate(rows, axis=0)


def _blockdiag(m, p):
    a, b = m.shape
    eye = jnp.eye(p, dtype=m.dtype)
    return (eye[:, None, :, None] * m[None, :, None, :]).reshape(p * a, p * b)


def kernel(knn_feature, W1, b1, W2, b2, W3, b3, W4, b4):
    B, N, K, C = knn_feature.shape
    topk = 8
    P = _PACK
    rows = (B * N * K) // P

    Wd1 = _blockdiag(W1.T, P)
    Wd2 = _blockdiag(W2.T, P)
    Wd3 = _blockdiag(W3.T, P)
    Wd4 = _blockdiag(W4.T, P)
    bd1 = jnp.tile(b1, P)[None, :]
    bd2 = jnp.tile(b2, P)[None, :]
    bd3 = jnp.tile(b3, P)[None, :]
    bd4 = jnp.tile(b4, P)[None, :]

    X = knn_feature.reshape(rows, P * C)

    scores = pl.pallas_call(
        _mlp_kernel,
        grid=(rows // _TILE,),
        in_specs=[
            pl.BlockSpec((_TILE, P * C), lambda i: (i, 0)),
            pl.BlockSpec((P * C, P * 32), lambda i: (0, 0)),
            pl.BlockSpec((1, P * 32), lambda i: (0, 0)),
            pl.BlockSpec((P * 32, P * 16), lambda i: (0, 0)),
            pl.BlockSpec((1, P * 16), lambda i: (0, 0)),
            pl.BlockSpec((P * 16, P * 8), lambda i: (0, 0)),
            pl.BlockSpec((1, P * 8), lambda i: (0, 0)),
            pl.BlockSpec((P * 8, P * 1), lambda i: (0, 0)),
            pl.BlockSpec((1, P * 1), lambda i: (0, 0)),
        ],
        out_specs=pl.BlockSpec((_TILE, P), lambda i: (i, 0)),
        out_shape=jax.ShapeDtypeStruct((rows, P), jnp.float32),
    )(X, Wd1, bd1, Wd2, bd2, Wd3, bd3, Wd4, bd4)

    scores_t = scores.reshape(B, N, K).transpose(0, 2, 1)

    idx_t = pl.pallas_call(
        functools.partial(_softmax_topk_kernel, n=N, k=K, topk=topk),
        grid=(B,),
        in_specs=[pl.BlockSpec((1, K, N), lambda b: (b, 0, 0))],
        out_specs=pl.BlockSpec((1, topk, N), lambda b: (b, 0, 0)),
        out_shape=jax.ShapeDtypeStruct((B, topk, N), jnp.int32),
    )(scores_t)

    return idx_t.transpose(0, 2, 1)


# speedup vs baseline: 1.0002x; 1.0002x over previous
import functools

import jax
import jax.numpy as jnp
from jax.experimental import pallas as pl

_PACK = 16
_TILE = 2048


def _mlp_kernel(x_ref, w1_ref, b1_ref, w2_ref, b2_ref, w3_ref, b3_ref,
                w4_ref, b4_ref, o_ref):
    x = x_ref[...]
    h = jnp.dot(x, w1_ref[...], preferred_element_type=jnp.float32) + b1_ref[...]
    h = jnp.dot(h, w2_ref[...], preferred_element_type=jnp.float32) + b2_ref[...]
    h = jnp.maximum(h, 0.0)
    h = jnp.dot(h, w3_ref[...], preferred_element_type=jnp.float32) + b3_ref[...]
    h = jnp.maximum(h, 0.0)
    h = jnp.dot(h, w4_ref[...], preferred_element_type=jnp.float32) + b4_ref[...]
    o_ref[...] = jnp.maximum(h, 0.0)


def _softmax_topk_kernel(s_ref, idx_ref, *, n, k, topk):
    x = s_ref[0].T
    m = jnp.max(x, axis=1, keepdims=True)
    p = jnp.exp(x - m)
    denom = jnp.sum(p, axis=1, keepdims=True)
    probs = p / denom
    kiota = jax.lax.broadcasted_iota(jnp.int32, (k, n), 0)
    rows = []
    a = probs
    for _ in range(topk):
        mx = jnp.max(a, axis=0, keepdims=True)
        sel = jnp.min(jnp.where(a == mx, kiota, k), axis=0, keepdims=True)
        rows.append(sel)
        a = jnp.where(kiota == sel, -1.0, a)
    idx_ref[0] = jnp.concatenate(rows, axis=0)


def _blockdiag(m, p):
    a, b = m.shape
    eye = jnp.eye(p, dtype=m.dtype)
    return (eye[:, None, :, None] * m[None, :, None, :]).reshape(p * a, p * b)


def kernel(knn_feature, W1, b1, W2, b2, W3, b3, W4, b4):
    B, N, K, C = knn_feature.shape
    topk = 8
    P = _PACK
    rows = (B * N * K) // P

    Wd1 = _blockdiag(W1.T, P)
    Wd2 = _blockdiag(W2.T, P)
    Wd3 = _blockdiag(W3.T, P)
    Wd4 = _blockdiag(W4.T, P)
    bd1 = jnp.tile(b1, P)[None, :]
    bd2 = jnp.tile(b2, P)[None, :]
    bd3 = jnp.tile(b3, P)[None, :]
    bd4 = jnp.tile(b4, P)[None, :]

    X = knn_feature.reshape(rows, P * C)

    scores = pl.pallas_call(
        _mlp_kernel,
        grid=(rows // _TILE,),
        in_specs=[
            pl.BlockSpec((_TILE, P * C), lambda i: (i, 0)),
            pl.BlockSpec((P * C, P * 32), lambda i: (0, 0)),
            pl.BlockSpec((1, P * 32), lambda i: (0, 0)),
            pl.BlockSpec((P * 32, P * 16), lambda i: (0, 0)),
            pl.BlockSpec((1, P * 16), lambda i: (0, 0)),
            pl.BlockSpec((P * 16, P * 8), lambda i: (0, 0)),
            pl.BlockSpec((1, P * 8), lambda i: (0, 0)),
            pl.BlockSpec((P * 8, P * 1), lambda i: (0, 0)),
            pl.BlockSpec((1, P * 1), lambda i: (0, 0)),
        ],
        out_specs=pl.BlockSpec((_TILE, P), lambda i: (i, 0)),
        out_shape=jax.ShapeDtypeStruct((rows, P), jnp.float32),
    )(X, Wd1, bd1, Wd2, bd2, Wd3, bd3, Wd4, bd4)

    idx_t = pl.pallas_call(
        functools.partial(_softmax_topk_kernel, n=N, k=K, topk=topk),
        grid=(B,),
        in_specs=[pl.BlockSpec((1, N, K), lambda b: (b, 0, 0))],
        out_specs=pl.BlockSpec((1, topk, N), lambda b: (b, 0, 0)),
        out_shape=jax.ShapeDtypeStruct((B, topk, N), jnp.int32),
    )(scores.reshape(B, N, K))

    return idx_t.transpose(0, 2, 1)


# speedup vs baseline: 5.7255x; 5.7244x over previous
import functools

import jax
import jax.numpy as jnp
from jax.experimental import pallas as pl


def _mlp_kernel(x_ref, w1_ref, b1_ref, w2_ref, b2_ref, w3_ref, b3_ref,
                w4_ref, b4_ref, o_ref):
    x = x_ref[0, 0]
    h = jnp.dot(w1_ref[...], x, preferred_element_type=jnp.float32) + b1_ref[...]
    h = jnp.dot(w2_ref[...], h, preferred_element_type=jnp.float32) + b2_ref[...]
    h = jnp.maximum(h, 0.0)
    h = jnp.dot(w3_ref[...], h, preferred_element_type=jnp.float32) + b3_ref[...]
    h = jnp.maximum(h, 0.0)
    h = jnp.dot(w4_ref[...], h, preferred_element_type=jnp.float32) + b4_ref[...]
    o_ref[0, 0] = jnp.maximum(h, 0.0)


def _softmax_topk_kernel(s_ref, idx_ref, *, n, k, topk):
    x = s_ref[0]
    m = jnp.max(x, axis=1, keepdims=True)
    p = jnp.exp(x - m)
    denom = jnp.sum(p, axis=1, keepdims=True)
    probs = p / denom
    kiota = jax.lax.broadcasted_iota(jnp.int32, (k, n), 0)
    rows = []
    a = probs
    for _ in range(topk):
        mx = jnp.max(a, axis=0, keepdims=True)
        sel = jnp.min(jnp.where(a == mx, kiota, k), axis=0, keepdims=True)
        rows.append(sel)
        a = jnp.where(kiota == sel, -1.0, a)
    idx_ref[0] = jnp.concatenate(rows, axis=0)


def kernel(knn_feature, W1, b1, W2, b2, W3, b3, W4, b4):
    B, N, K, C = knn_feature.shape
    topk = 8

    Xt = knn_feature.transpose(0, 2, 3, 1)

    scores_t = pl.pallas_call(
        _mlp_kernel,
        grid=(B, K),
        in_specs=[
            pl.BlockSpec((1, 1, C, N), lambda b, k: (b, k, 0, 0)),
            pl.BlockSpec(W1.shape, lambda b, k: (0, 0)),
            pl.BlockSpec((32, 1), lambda b, k: (0, 0)),
            pl.BlockSpec(W2.shape, lambda b, k: (0, 0)),
            pl.BlockSpec((16, 1), lambda b, k: (0, 0)),
            pl.BlockSpec(W3.shape, lambda b, k: (0, 0)),
            pl.BlockSpec((8, 1), lambda b, k: (0, 0)),
            pl.BlockSpec(W4.shape, lambda b, k: (0, 0)),
            pl.BlockSpec((1, 1), lambda b, k: (0, 0)),
        ],
        out_specs=pl.BlockSpec((1, 1, 1, N), lambda b, k: (b, k, 0, 0)),
        out_shape=jax.ShapeDtypeStruct((B, K, 1, N), jnp.float32),
    )(Xt, W1, b1[:, None], W2, b2[:, None], W3, b3[:, None], W4, b4[:, None])
    scores_t = scores_t.reshape(B, K, N)

    idx_t = pl.pallas_call(
        functools.partial(_softmax_topk_kernel, n=N, k=K, topk=topk),
        grid=(B,),
        in_specs=[pl.BlockSpec((1, K, N), lambda b: (b, 0, 0))],
        out_specs=pl.BlockSpec((1, topk, N), lambda b: (b, 0, 0)),
        out_shape=jax.ShapeDtypeStruct((B, topk, N), jnp.int32),
    )(scores_t)

    return idx_t.transpose(0, 2, 1)
